# baseline (speedup 1.0000x reference)
import functools

import jax
import jax.numpy as jnp
from jax import lax
from jax.experimental import pallas as pl
from jax.experimental.pallas import tpu as pltpu
from jax.experimental.pallas import tpu_sc as plsc

_NUM_CLASSES = 133
_POS_WEIGHT = 1.0
_B, _Q, _P, _C, _G = 2, 300, 16384, 133, 50


def _cost_kernel(use_cls_ref, pm_ref, pl_ref, gm_ref, gl_ref, cost_ref):
    pm = jax.nn.sigmoid(pm_ref[0])
    gm = gm_ref[0]
    numer = 2.0 * lax.dot_general(
        pm, gm, (((1,), (1,)), ((), ())),
        preferred_element_type=jnp.float32)
    denom = jnp.sum(pm, axis=1, keepdims=True) + \
        jnp.sum(gm, axis=1, keepdims=True).reshape(1, _G)
    dice = 1.0 - (numer + 1.0) / (denom + 1.0)
    scores = jax.nn.softmax(pl_ref[0], axis=-1)
    gl = gl_ref[0]
    onehot = (gl == lax.broadcasted_iota(jnp.int32, (_C, _G), 0)
              ).astype(jnp.float32)
    cls_cost = -jnp.dot(scores, onehot, preferred_element_type=jnp.float32)
    cost_ref[0] = dice + use_cls_ref[0, 0] * cls_cost


def _assign_kernel(cost_ref, gl_ref, labels_ref, oh_ref, asg_ref,
                   tidx_ref, widx_ref):
    b = pl.program_id(0)
    cost = cost_ref[0]
    iota_q = lax.broadcasted_iota(jnp.int32, (_Q, 1), 0)
    iota_g = lax.broadcasted_iota(jnp.int32, (_Q, _G), 1)

    def body(g, carry):
        taken, gidx = carry
        col = jnp.sum(jnp.where(iota_g == g, cost, 0.0), axis=1,
                      keepdims=True)
        val = col + taken * 2e9
        m = jnp.min(val)
        q = jnp.min(jnp.where(val == m, iota_q, _Q))
        sel = iota_q == q
        taken = jnp.where(sel, 1.0, taken)
        gidx = jnp.where(sel, g, gidx)
        return taken, gidx

    taken, gidx = lax.fori_loop(
        0, _G, body,
        (jnp.zeros((_Q, 1), jnp.float32), jnp.full((_Q, 1), -1, jnp.int32)))

    oh = (gidx == lax.broadcasted_iota(jnp.int32, (_Q, _G), 1))
    gl = gl_ref[0]
    lab = jnp.sum(jnp.where(oh, gl, 0), axis=1, keepdims=True)
    lab = jnp.where(taken > 0, lab, _NUM_CLASSES)
    iota8 = lax.broadcasted_iota(jnp.int32, (1, 8), 1)
    labels_ref[0] = jnp.broadcast_to(lab, (_Q, 8))
    oh_ref[0] = oh.astype(jnp.float32)
    asg_ref[0] = jnp.broadcast_to(taken, (_Q, 8))
    tbase = jnp.where(taken > 0, b * _G + gidx, 2 * _G)
    wbase = jnp.where(taken > 0, 2 * _G + 1, 2 * _G)
    tidx_ref[0] = tbase * 8 + iota8
    widx_ref[0] = wbase * 8 + iota8


_SR = 2048
_SPR = _P // _SR
_NROWS = _B * _Q * _SPR
_NGRP = _NROWS // 16
_NW = 32
_ITERS = (2 * _NGRP + _NW - 1) // _NW


def _sc_build_body(table_ref, tidx_ref, widx_ref, mt_ref, mw_ref,
                   idx_v, rows_v, sem):
    wid = lax.axis_index("s") * 2 + lax.axis_index("c")

    def gather_group(idx_hbm, out_hbm, grp):
        pltpu.sync_copy(idx_hbm.at[pl.ds(grp * 16, 16)], idx_v)
        pltpu.async_copy(table_ref.at[idx_v], rows_v, sem).wait()
        pltpu.sync_copy(rows_v, out_hbm.at[pl.ds(grp * 16, 16)])

    for i in range(_ITERS):
        g = wid + _NW * i

        @pl.when(g < _NGRP)
        def _():
            gather_group(tidx_ref, mt_ref, g)

        @pl.when(jnp.logical_and(g >= _NGRP, g < 2 * _NGRP))
        def _():
            gather_group(widx_ref, mw_ref, g - _NGRP)


def kernel(pred_masks, pred_labels, gt_masks, gt_labels, layer):
    use_cls = jnp.where(layer == 0, 0.0, 1.0).astype(jnp.float32)
    use_cls = use_cls.reshape(1, 1)
    gl3 = gt_labels.astype(jnp.int32).reshape(_B, 1, _G)

    cost = pl.pallas_call(
        _cost_kernel,
        grid=(_B,),
        in_specs=[
            pl.BlockSpec((1, 1), lambda b: (0, 0),
                         memory_space=pltpu.SMEM),
            pl.BlockSpec((1, _Q, _P), lambda b: (b, 0, 0)),
            pl.BlockSpec((1, _Q, _C), lambda b: (b, 0, 0)),
            pl.BlockSpec((1, _G, _P), lambda b: (b, 0, 0)),
            pl.BlockSpec((1, 1, _G), lambda b: (b, 0, 0)),
        ],
        out_specs=pl.BlockSpec((1, _Q, _G), lambda b: (b, 0, 0)),
        out_shape=jax.ShapeDtypeStruct((_B, _Q, _G), jnp.float32),
    )(use_cls, pred_masks, pred_labels, gt_masks, gl3)

    labels8, oh, asg8, tidx8, widx8 = pl.pallas_call(
        _assign_kernel,
        grid=(_B,),
        in_specs=[
            pl.BlockSpec((1, _Q, _G), lambda b: (b, 0, 0)),
            pl.BlockSpec((1, 1, _G), lambda b: (b, 0, 0)),
        ],
        out_specs=[
            pl.BlockSpec((1, _Q, 8), lambda b: (b, 0, 0)),
            pl.BlockSpec((1, _Q, _G), lambda b: (b, 0, 0)),
            pl.BlockSpec((1, _Q, 8), lambda b: (b, 0, 0)),
            pl.BlockSpec((1, _Q, 8), lambda b: (b, 0, 0)),
            pl.BlockSpec((1, _Q, 8), lambda b: (b, 0, 0)),
        ],
        out_shape=[
            jax.ShapeDtypeStruct((_B, _Q, 8), jnp.int32),
            jax.ShapeDtypeStruct((_B, _Q, _G), jnp.float32),
            jax.ShapeDtypeStruct((_B, _Q, 8), jnp.float32),
            jax.ShapeDtypeStruct((_B, _Q, 8), jnp.int32),
            jax.ShapeDtypeStruct((_B, _Q, 8), jnp.int32),
        ],
    )(cost, gl3)

    table = jnp.concatenate(
        [gt_masks.reshape(_B * _G, _P),
         jnp.zeros((1, _P), jnp.float32),
         jnp.ones((1, _P), jnp.float32)], axis=0).reshape(-1, _SR)

    sc_build = functools.partial(
        pl.kernel,
        mesh=plsc.VectorSubcoreMesh(core_axis_name="c", subcore_axis_name="s"),
        out_type=[
            jax.ShapeDtypeStruct((_NROWS, _SR), jnp.float32),
            jax.ShapeDtypeStruct((_NROWS, _SR), jnp.float32),
        ],
        scratch_types=[
            pltpu.VMEM((16,), jnp.int32),
            pltpu.VMEM((16, _SR), jnp.float32),
            pltpu.SemaphoreType.DMA,
        ],
    )(_sc_build_body)
    mt_r, mw_r = sc_build(table, tidx8.reshape(_NROWS), widx8.reshape(_NROWS))
    mask_targets = mt_r.reshape(_B, _Q, _P)
    mask_weights = mw_r.reshape(_B, _Q, _P)

    labels = labels8[..., 0]
    label_weights = jnp.ones((_B, _Q, _C), jnp.float32)
    return (pred_masks, pred_labels, labels, label_weights,
            mask_targets, mask_weights)


# speedup vs baseline: 1.0037x; 1.0037x over previous
import functools

import jax
import jax.numpy as jnp
from jax import lax
from jax.experimental import pallas as pl
from jax.experimental.pallas import tpu as pltpu
from jax.experimental.pallas import tpu_sc as plsc

_NUM_CLASSES = 133
_POS_WEIGHT = 1.0
_B, _Q, _P, _C, _G = 2, 300, 16384, 133, 50


def _cost_kernel(use_cls_ref, pm_ref, pl_ref, gm_ref, gl_ref, cost_ref):
    pm = jax.nn.sigmoid(pm_ref[0])
    gm = gm_ref[0]
    numer = 2.0 * lax.dot_general(
        pm, gm, (((1,), (1,)), ((), ())),
        preferred_element_type=jnp.float32)
    denom = jnp.sum(pm, axis=1, keepdims=True) + \
        jnp.sum(gm, axis=1, keepdims=True).reshape(1, _G)
    dice = 1.0 - (numer + 1.0) / (denom + 1.0)
    scores = jax.nn.softmax(pl_ref[0], axis=-1)
    gl = gl_ref[0]
    onehot = (gl == lax.broadcasted_iota(jnp.int32, (_C, _G), 0)
              ).astype(jnp.float32)
    cls_cost = -jnp.dot(scores, onehot, preferred_element_type=jnp.float32)
    cost_ref[0] = dice + use_cls_ref[0, 0] * cls_cost


def _assign_kernel(cost_ref, gl_ref, labels_ref, oh_ref, asg_ref,
                   tidx_ref, widx_ref):
    b = pl.program_id(0)
    cost = cost_ref[0]
    iota_q = lax.broadcasted_iota(jnp.int32, (_Q, 1), 0)
    iota_g = lax.broadcasted_iota(jnp.int32, (_Q, _G), 1)

    def body(g, carry):
        taken, gidx = carry
        col = jnp.sum(jnp.where(iota_g == g, cost, 0.0), axis=1,
                      keepdims=True)
        val = col + taken * 2e9
        m = jnp.min(val)
        q = jnp.min(jnp.where(val == m, iota_q, _Q))
        sel = iota_q == q
        taken = jnp.where(sel, 1.0, taken)
        gidx = jnp.where(sel, g, gidx)
        return taken, gidx

    taken, gidx = lax.fori_loop(
        0, _G, body,
        (jnp.zeros((_Q, 1), jnp.float32), jnp.full((_Q, 1), -1, jnp.int32)))

    oh = (gidx == lax.broadcasted_iota(jnp.int32, (_Q, _G), 1))
    gl = gl_ref[0]
    lab = jnp.sum(jnp.where(oh, gl, 0), axis=1, keepdims=True)
    lab = jnp.where(taken > 0, lab, _NUM_CLASSES)
    iota8 = lax.broadcasted_iota(jnp.int32, (1, 8), 1)
    labels_ref[0] = jnp.broadcast_to(lab, (_Q, 8))
    oh_ref[0] = oh.astype(jnp.float32)
    asg_ref[0] = jnp.broadcast_to(taken, (_Q, 8))
    tbase = jnp.where(taken > 0, b * _G + gidx, 2 * _G)
    wbase = jnp.where(taken > 0, 2 * _G + 1, 2 * _G)
    tidx_ref[0] = tbase * 8 + iota8
    widx_ref[0] = wbase * 8 + iota8


_SR = 2048
_SPR = _P // _SR
_NROWS = _B * _Q * _SPR
_GS = 16
_NGRP = _NROWS // _GS
_NW = 32
_ITERS = (2 * _NGRP + _NW - 1) // _NW


def _sc_build_body(table_ref, idx_ref, mt_ref, mw_ref,
                   idx_v, rows_v, sem_in, sem_out):
    wid = lax.axis_index("s") * 2 + lax.axis_index("c")

    def grp(k):
        return wid + _NW * k

    def valid(k):
        return grp(k) < 2 * _NGRP

    def issue_gather(k, slot):
        g = grp(k)
        pltpu.sync_copy(idx_ref.at[pl.ds(g * _GS, _GS)], idx_v.at[slot])
        pltpu.async_copy(
            table_ref.at[idx_v.at[slot]], rows_v.at[slot], sem_in.at[slot])

    def wait_gather(slot):
        pltpu.make_async_copy(
            table_ref.at[idx_v.at[slot]], rows_v.at[slot],
            sem_in.at[slot]).wait()

    def issue_out(k, slot):
        g = grp(k)

        @pl.when(g < _NGRP)
        def _():
            pltpu.async_copy(
                rows_v.at[slot], mt_ref.at[pl.ds(g * _GS, _GS)],
                sem_out.at[slot])

        @pl.when(g >= _NGRP)
        def _():
            pltpu.async_copy(
                rows_v.at[slot], mw_ref.at[pl.ds((g - _NGRP) * _GS, _GS)],
                sem_out.at[slot])

    def wait_out(slot):
        pltpu.make_async_copy(
            rows_v.at[slot], mt_ref.at[pl.ds(0, _GS)],
            sem_out.at[slot]).wait()

    @pl.when(valid(0))
    def _():
        issue_gather(0, 0)
    for k in range(_ITERS):
        slot = k % 2
        nslot = 1 - slot
        if k + 1 < _ITERS:
            if k >= 1:
                @pl.when(valid(k - 1))
                def _():
                    wait_out(nslot)

            @pl.when(valid(k + 1))
            def _():
                issue_gather(k + 1, nslot)

        @pl.when(valid(k))
        def _():
            wait_gather(slot)
            issue_out(k, slot)
    for k in (_ITERS - 2, _ITERS - 1):
        @pl.when(valid(k))
        def _():
            wait_out(k % 2)


def kernel(pred_masks, pred_labels, gt_masks, gt_labels, layer):
    use_cls = jnp.where(layer == 0, 0.0, 1.0).astype(jnp.float32)
    use_cls = use_cls.reshape(1, 1)
    gl3 = gt_labels.astype(jnp.int32).reshape(_B, 1, _G)

    cost = pl.pallas_call(
        _cost_kernel,
        grid=(_B,),
        in_specs=[
            pl.BlockSpec((1, 1), lambda b: (0, 0),
                         memory_space=pltpu.SMEM),
            pl.BlockSpec((1, _Q, _P), lambda b: (b, 0, 0)),
            pl.BlockSpec((1, _Q, _C), lambda b: (b, 0, 0)),
            pl.BlockSpec((1, _G, _P), lambda b: (b, 0, 0)),
            pl.BlockSpec((1, 1, _G), lambda b: (b, 0, 0)),
        ],
        out_specs=pl.BlockSpec((1, _Q, _G), lambda b: (b, 0, 0)),
        out_shape=jax.ShapeDtypeStruct((_B, _Q, _G), jnp.float32),
    )(use_cls, pred_masks, pred_labels, gt_masks, gl3)

    labels8, oh, asg8, tidx8, widx8 = pl.pallas_call(
        _assign_kernel,
        grid=(_B,),
        in_specs=[
            pl.BlockSpec((1, _Q, _G), lambda b: (b, 0, 0)),
            pl.BlockSpec((1, 1, _G), lambda b: (b, 0, 0)),
        ],
        out_specs=[
            pl.BlockSpec((1, _Q, 8), lambda b: (b, 0, 0)),
            pl.BlockSpec((1, _Q, _G), lambda b: (b, 0, 0)),
            pl.BlockSpec((1, _Q, 8), lambda b: (b, 0, 0)),
            pl.BlockSpec((1, _Q, 8), lambda b: (b, 0, 0)),
            pl.BlockSpec((1, _Q, 8), lambda b: (b, 0, 0)),
        ],
        out_shape=[
            jax.ShapeDtypeStruct((_B, _Q, 8), jnp.int32),
            jax.ShapeDtypeStruct((_B, _Q, _G), jnp.float32),
            jax.ShapeDtypeStruct((_B, _Q, 8), jnp.float32),
            jax.ShapeDtypeStruct((_B, _Q, 8), jnp.int32),
            jax.ShapeDtypeStruct((_B, _Q, 8), jnp.int32),
        ],
    )(cost, gl3)

    table = jnp.concatenate(
        [gt_masks.reshape(_B * _G, _P),
         jnp.zeros((1, _P), jnp.float32),
         jnp.ones((1, _P), jnp.float32)], axis=0).reshape(-1, _SR)

    sc_build = functools.partial(
        pl.kernel,
        mesh=plsc.VectorSubcoreMesh(core_axis_name="c", subcore_axis_name="s"),
        out_type=[
            jax.ShapeDtypeStruct((_NROWS, _SR), jnp.float32),
            jax.ShapeDtypeStruct((_NROWS, _SR), jnp.float32),
        ],
        scratch_types=[
            pltpu.VMEM((2, _GS), jnp.int32),
            pltpu.VMEM((2, _GS, _SR), jnp.float32),
            pltpu.SemaphoreType.DMA((2,)),
            pltpu.SemaphoreType.DMA((2,)),
        ],
    )(_sc_build_body)
    idx_all = jnp.concatenate(
        [tidx8.reshape(_NROWS), widx8.reshape(_NROWS)], axis=0)
    mt_r, mw_r = sc_build(table, idx_all)
    mask_targets = mt_r.reshape(_B, _Q, _P)
    mask_weights = mw_r.reshape(_B, _Q, _P)

    labels = labels8[..., 0]
    label_weights = jnp.ones((_B, _Q, _C), jnp.float32)
    return (pred_masks, pred_labels, labels, label_weights,
            mask_targets, mask_weights)
